# baseline (speedup 1.0000x reference)
import jax
import jax.numpy as jnp
from jax import lax
from jax.experimental import pallas as pl
from jax.experimental.pallas import tpu as pltpu
from jax.experimental.pallas import tpu_sc as plsc

B = 8
MAX_SEQLEN = 2048
TOTAL_TOK = 8192
D = 1024

NC = 2
NS = 16
NW = NC * NS
TOTAL_ROWS = B * MAX_SEQLEN
CHUNK = 32
CPB = MAX_SEQLEN // CHUNK
NCHUNK = TOTAL_ROWS // CHUNK // NW
LANES = 16


def _body(flat_hbm, cu_hbm, z_hbm, out_hbm, cu_v, zbuf, buf0, buf1,
          idx_v, sg0, sg1, ss0, ss1, sz):
    wid = lax.axis_index("s") * NC + lax.axis_index("c")

    pltpu.sync_copy(cu_hbm, cu_v)
    pltpu.sync_copy(z_hbm, zbuf)

    cu_vec = cu_v[...]
    iota = lax.broadcasted_iota(jnp.int32, (LANES,), 0)

    nv = []
    for t in range(NCHUNK):
        g = wid + t * NW
        bt = g // CPB
        jc = (g % CPB) * CHUNK
        st = jnp.sum(jnp.where(iota == bt, cu_vec, 0))
        en = jnp.sum(jnp.where(iota == bt + 1, cu_vec, 0))
        ln = jnp.minimum(en - st, MAX_SEQLEN)
        nv.append(jnp.clip(ln - jc, 0, CHUNK))
        for h in range(0, CHUNK, LANES):
            idx_v[t, pl.ds(h, LANES)] = jnp.minimum(
                st + jc + h + iota, TOTAL_TOK - 1)

    bufs = (buf0, buf1)
    sgs = (sg0, sg1)
    sss = (ss0, ss1)

    def out_chunk(t):
        g = wid + t * NW
        return out_hbm.at[pl.ds(pl.multiple_of(g * CHUNK, CHUNK), CHUNK)]

    for t in range(NCHUNK):
        @pl.when(nv[t] == 0)
        def _():
            pltpu.async_copy(zbuf, out_chunk(t), sz)

    def fix_and_scatter(t):
        p = t & 1
        pltpu.make_async_copy(
            flat_hbm.at[idx_v.at[t]], bufs[p], sgs[p]).wait()

        def zero_row(r, carry):
            for i in range(D // LANES):
                bufs[p][r, pl.ds(i * LANES, LANES)] = jnp.zeros(
                    (LANES,), jnp.float32)
            return carry

        lax.fori_loop(nv[t], CHUNK, zero_row, 0)
        pltpu.async_copy(bufs[p], out_chunk(t), sss[p])

    for t in range(NCHUNK):
        p = t & 1
        if t >= 2:
            @pl.when(nv[t - 2] > 0)
            def _():
                pltpu.make_async_copy(bufs[p], out_chunk(t - 2), sss[p]).wait()

        @pl.when(nv[t] > 0)
        def _():
            pltpu.async_copy(flat_hbm.at[idx_v.at[t]], bufs[p], sgs[p])

        if t >= 1:
            @pl.when(nv[t - 1] > 0)
            def _():
                fix_and_scatter(t - 1)

    @pl.when(nv[NCHUNK - 1] > 0)
    def _():
        fix_and_scatter(NCHUNK - 1)

    for t in (NCHUNK - 2, NCHUNK - 1):
        @pl.when(nv[t] > 0)
        def _():
            pltpu.make_async_copy(bufs[t & 1], out_chunk(t), sss[t & 1]).wait()

    for t in range(NCHUNK):
        @pl.when(nv[t] == 0)
        def _():
            pltpu.make_async_copy(zbuf, out_chunk(t), sz).wait()


@jax.jit
def kernel(flat, cu_seqlens):
    cu_pad = jnp.concatenate(
        [cu_seqlens.astype(jnp.int32), jnp.zeros((16 - (B + 1),), jnp.int32)])
    zeros_chunk = jnp.zeros((CHUNK, D), jnp.float32)

    mesh = plsc.VectorSubcoreMesh(core_axis_name="c", subcore_axis_name="s")
    out = pl.kernel(
        _body,
        out_type=jax.ShapeDtypeStruct((TOTAL_ROWS, D), jnp.float32),
        mesh=mesh,
        scratch_types=[
            pltpu.VMEM((LANES,), jnp.int32),
            pltpu.VMEM((CHUNK, D), jnp.float32),
            pltpu.VMEM((CHUNK, D), jnp.float32),
            pltpu.VMEM((CHUNK, D), jnp.float32),
            pltpu.VMEM((NCHUNK, CHUNK), jnp.int32),
            pltpu.SemaphoreType.DMA,
            pltpu.SemaphoreType.DMA,
            pltpu.SemaphoreType.DMA,
            pltpu.SemaphoreType.DMA,
            pltpu.SemaphoreType.DMA,
        ],
        compiler_params=pltpu.CompilerParams(needs_layout_passes=False),
    )(flat, cu_pad, zeros_chunk)
    return out.reshape(B, MAX_SEQLEN, D)


# speedup vs baseline: 1.0535x; 1.0535x over previous
import jax
import jax.numpy as jnp
from jax import lax
from jax.experimental import pallas as pl
from jax.experimental.pallas import tpu as pltpu
from jax.experimental.pallas import tpu_sc as plsc

B = 8
MAX_SEQLEN = 2048
TOTAL_TOK = 8192
D = 1024

NC = 2
NS = 16
NW = NC * NS
TOTAL_ROWS = B * MAX_SEQLEN
RPW = TOTAL_ROWS // NW
CHUNK = 32
NCHUNK = RPW // CHUNK
NB = 3
ZROWS = 16
LANES = 16


def _body(flat_hbm, cu_hbm, z_hbm, out_hbm, cu_v, zbuf, buf0, buf1, buf2,
          idx_v, idxb_v, sg0, sg1, sg2, ss0, ss1, ss2, sz):
    c = lax.axis_index("c")
    s = lax.axis_index("s")
    b = s // 2
    q = 2 * (s % 2) + ((b + c) % 2)
    j0 = q * RPW
    row0 = b * MAX_SEQLEN + j0

    pltpu.sync_copy(cu_hbm, cu_v)
    pltpu.sync_copy(z_hbm, zbuf)

    cu_vec = cu_v[...]
    iota = lax.broadcasted_iota(jnp.int32, (LANES,), 0)
    start = jnp.sum(jnp.where(iota == b, cu_vec, 0))
    end = jnp.sum(jnp.where(iota == b + 1, cu_vec, 0))
    seg_len = jnp.minimum(end - start, MAX_SEQLEN)
    nvalid = jnp.clip(seg_len - j0, 0, RPW)
    kfull = nvalid // CHUNK
    rem = nvalid % CHUNK
    kzero = kfull + (rem > 0).astype(jnp.int32)

    base = start + j0
    bufs = (buf0, buf1, buf2)
    sgs = (sg0, sg1, sg2)
    sss = (ss0, ss1, ss2)

    def out_half(k, h):
        return out_hbm.at[
            pl.ds(pl.multiple_of(row0 + k * CHUNK + h, ZROWS), ZROWS)]

    def out_chunk(k):
        return out_hbm.at[pl.ds(pl.multiple_of(row0 + k * CHUNK, CHUNK), CHUNK)]

    for k in range(NCHUNK):
        @pl.when(k >= kzero)
        def _():
            pltpu.async_copy(zbuf, out_half(k, 0), sz)
            pltpu.async_copy(zbuf, out_half(k, ZROWS), sz)

    for k in range(NCHUNK):
        p = k % NB
        if k >= NB:
            @pl.when(k - NB < kfull)
            def _():
                pltpu.make_async_copy(
                    bufs[p], out_chunk(k - NB), sss[p]).wait()

        for h in range(0, CHUNK, LANES):
            idx_v[k, pl.ds(h, LANES)] = jnp.minimum(
                base + (k * CHUNK + h) + iota, TOTAL_TOK - 1)

        @pl.when(k < kfull)
        def _():
            pltpu.async_copy(flat_hbm.at[idx_v.at[k]], bufs[p], sgs[p])

        if k >= 1:
            q1 = (k - 1) % NB

            @pl.when(k - 1 < kfull)
            def _():
                pltpu.make_async_copy(
                    flat_hbm.at[idx_v.at[k - 1]], bufs[q1], sgs[q1]).wait()
                pltpu.async_copy(bufs[q1], out_chunk(k - 1), sss[q1])

    @pl.when(NCHUNK - 1 < kfull)
    def _():
        q1 = (NCHUNK - 1) % NB
        pltpu.make_async_copy(
            flat_hbm.at[idx_v.at[NCHUNK - 1]], bufs[q1], sgs[q1]).wait()
        pltpu.async_copy(bufs[q1], out_chunk(NCHUNK - 1), sss[q1])

    for k in range(NCHUNK - NB, NCHUNK):
        @pl.when(k < kfull)
        def _():
            pltpu.make_async_copy(
                bufs[k % NB], out_chunk(k), sss[k % NB]).wait()

    bbase = base + kfull * CHUNK
    for h in range(0, CHUNK, LANES):
        idxb_v[pl.ds(h, LANES)] = jnp.minimum(bbase + h + iota, TOTAL_TOK - 1)

    @pl.when(rem > 0)
    def _():
        pltpu.async_copy(flat_hbm.at[idxb_v], buf0, sg0)
        pltpu.make_async_copy(flat_hbm.at[idxb_v], buf0, sg0).wait()

        def zero_row(r, carry):
            for i in range(D // LANES):
                buf0[r, pl.ds(i * LANES, LANES)] = jnp.zeros(
                    (LANES,), jnp.float32)
            return carry

        lax.fori_loop(rem, CHUNK, zero_row, 0)

    bchunk_dst = out_hbm.at[
        pl.ds(pl.multiple_of(row0 + kfull * CHUNK, CHUNK), CHUNK)]

    @pl.when(rem > 0)
    def _():
        pltpu.async_copy(buf0, bchunk_dst, sz)

    for k in range(NCHUNK):
        @pl.when(k >= kzero)
        def _():
            pltpu.make_async_copy(zbuf, out_half(k, 0), sz).wait()
            pltpu.make_async_copy(zbuf, out_half(k, ZROWS), sz).wait()

    @pl.when(rem > 0)
    def _():
        pltpu.make_async_copy(buf0, bchunk_dst, sz).wait()


@jax.jit
def kernel(flat, cu_seqlens):
    cu_pad = jnp.concatenate(
        [cu_seqlens.astype(jnp.int32), jnp.zeros((16 - (B + 1),), jnp.int32)])
    zeros_rows = jnp.zeros((ZROWS, D), jnp.float32)

    mesh = plsc.VectorSubcoreMesh(core_axis_name="c", subcore_axis_name="s")
    out = pl.kernel(
        _body,
        out_type=jax.ShapeDtypeStruct((TOTAL_ROWS, D), jnp.float32),
        mesh=mesh,
        scratch_types=[
            pltpu.VMEM((LANES,), jnp.int32),
            pltpu.VMEM((ZROWS, D), jnp.float32),
            pltpu.VMEM((CHUNK, D), jnp.float32),
            pltpu.VMEM((CHUNK, D), jnp.float32),
            pltpu.VMEM((CHUNK, D), jnp.float32),
            pltpu.VMEM((NCHUNK, CHUNK), jnp.int32),
            pltpu.VMEM((CHUNK,), jnp.int32),
            pltpu.SemaphoreType.DMA,
            pltpu.SemaphoreType.DMA,
            pltpu.SemaphoreType.DMA,
            pltpu.SemaphoreType.DMA,
            pltpu.SemaphoreType.DMA,
            pltpu.SemaphoreType.DMA,
            pltpu.SemaphoreType.DMA,
        ],
        compiler_params=pltpu.CompilerParams(needs_layout_passes=False),
    )(flat, cu_pad, zeros_rows)
    return out.reshape(B, MAX_SEQLEN, D)


# speedup vs baseline: 1.1036x; 1.0476x over previous
import jax
import jax.numpy as jnp
from jax import lax
from jax.experimental import pallas as pl
from jax.experimental.pallas import tpu as pltpu
from jax.experimental.pallas import tpu_sc as plsc

B = 8
MAX_SEQLEN = 2048
TOTAL_TOK = 8192
D = 1024

NC = 2
NS = 16
NW = NC * NS
TOTAL_ROWS = B * MAX_SEQLEN
RPW = TOTAL_ROWS // NW
CHUNK = 32
NCHUNK = RPW // CHUNK
NB = 3
ZROWS = 16
LANES = 16


def _body(flat_hbm, cu_hbm, z_hbm, out_hbm, cu_v, zbuf, buf0, buf1, buf2,
          idx_v, idxb_v, sg0, sg1, sg2, ss0, ss1, ss2, sz):
    c = lax.axis_index("c")
    s = lax.axis_index("s")
    b = s // 2
    q = 2 * (s % 2) + ((b + c) % 2)
    j0 = q * RPW
    row0 = b * MAX_SEQLEN + j0

    pltpu.sync_copy(cu_hbm, cu_v.at[pl.ds(0, B + 1)])
    pltpu.sync_copy(z_hbm, zbuf)

    cu_vec = cu_v[...]
    iota = lax.broadcasted_iota(jnp.int32, (LANES,), 0)
    start = jnp.sum(jnp.where(iota == b, cu_vec, 0))
    end = jnp.sum(jnp.where(iota == b + 1, cu_vec, 0))
    seg_len = jnp.minimum(end - start, MAX_SEQLEN)
    nvalid = jnp.clip(seg_len - j0, 0, RPW)
    kfull = nvalid // CHUNK
    rem = nvalid % CHUNK
    kzero = kfull + (rem > 0).astype(jnp.int32)

    base = start + j0
    bufs = (buf0, buf1, buf2)
    sgs = (sg0, sg1, sg2)
    sss = (ss0, ss1, ss2)

    def out_half(k, h):
        return out_hbm.at[
            pl.ds(pl.multiple_of(row0 + k * CHUNK + h, ZROWS), ZROWS)]

    def out_chunk(k):
        return out_hbm.at[pl.ds(pl.multiple_of(row0 + k * CHUNK, CHUNK), CHUNK)]

    for k in range(NCHUNK):
        @pl.when(k >= kzero)
        def _():
            pltpu.async_copy(zbuf, out_half(k, 0), sz)
            pltpu.async_copy(zbuf, out_half(k, ZROWS), sz)

    for k in range(NCHUNK):
        p = k % NB
        if k >= NB:
            @pl.when(k - NB < kfull)
            def _():
                pltpu.make_async_copy(
                    bufs[p], out_chunk(k - NB), sss[p]).wait()

        for h in range(0, CHUNK, LANES):
            idx_v[k, pl.ds(h, LANES)] = jnp.minimum(
                base + (k * CHUNK + h) + iota, TOTAL_TOK - 1)

        @pl.when(k < kfull)
        def _():
            pltpu.async_copy(flat_hbm.at[idx_v.at[k]], bufs[p], sgs[p])

        if k >= 1:
            q1 = (k - 1) % NB

            @pl.when(k - 1 < kfull)
            def _():
                pltpu.make_async_copy(
                    flat_hbm.at[idx_v.at[k - 1]], bufs[q1], sgs[q1]).wait()
                pltpu.async_copy(bufs[q1], out_chunk(k - 1), sss[q1])

    @pl.when(NCHUNK - 1 < kfull)
    def _():
        q1 = (NCHUNK - 1) % NB
        pltpu.make_async_copy(
            flat_hbm.at[idx_v.at[NCHUNK - 1]], bufs[q1], sgs[q1]).wait()
        pltpu.async_copy(bufs[q1], out_chunk(NCHUNK - 1), sss[q1])

    for k in range(NCHUNK - NB, NCHUNK):
        @pl.when(k < kfull)
        def _():
            pltpu.make_async_copy(
                bufs[k % NB], out_chunk(k), sss[k % NB]).wait()

    bbase = base + kfull * CHUNK
    for h in range(0, CHUNK, LANES):
        idxb_v[pl.ds(h, LANES)] = jnp.minimum(bbase + h + iota, TOTAL_TOK - 1)

    @pl.when(rem > 0)
    def _():
        pltpu.async_copy(flat_hbm.at[idxb_v], buf0, sg0)
        pltpu.make_async_copy(flat_hbm.at[idxb_v], buf0, sg0).wait()

        def zero_row(r, carry):
            for i in range(D // LANES):
                buf0[r, pl.ds(i * LANES, LANES)] = jnp.zeros(
                    (LANES,), jnp.float32)
            return carry

        lax.fori_loop(rem, CHUNK, zero_row, 0)

    bchunk_dst = out_hbm.at[
        pl.ds(pl.multiple_of(row0 + kfull * CHUNK, CHUNK), CHUNK)]

    @pl.when(rem > 0)
    def _():
        pltpu.async_copy(buf0, bchunk_dst, sz)

    for k in range(NCHUNK):
        @pl.when(k >= kzero)
        def _():
            pltpu.make_async_copy(zbuf, out_half(k, 0), sz).wait()
            pltpu.make_async_copy(zbuf, out_half(k, ZROWS), sz).wait()

    @pl.when(rem > 0)
    def _():
        pltpu.make_async_copy(buf0, bchunk_dst, sz).wait()


@jax.jit
def kernel(flat, cu_seqlens):
    zeros_rows = jnp.zeros((ZROWS, D), jnp.float32)

    mesh = plsc.VectorSubcoreMesh(core_axis_name="c", subcore_axis_name="s")
    out = pl.kernel(
        _body,
        out_type=jax.ShapeDtypeStruct((TOTAL_ROWS, D), jnp.float32),
        mesh=mesh,
        scratch_types=[
            pltpu.VMEM((LANES,), jnp.int32),
            pltpu.VMEM((ZROWS, D), jnp.float32),
            pltpu.VMEM((CHUNK, D), jnp.float32),
            pltpu.VMEM((CHUNK, D), jnp.float32),
            pltpu.VMEM((CHUNK, D), jnp.float32),
            pltpu.VMEM((NCHUNK, CHUNK), jnp.int32),
            pltpu.VMEM((CHUNK,), jnp.int32),
            pltpu.SemaphoreType.DMA,
            pltpu.SemaphoreType.DMA,
            pltpu.SemaphoreType.DMA,
            pltpu.SemaphoreType.DMA,
            pltpu.SemaphoreType.DMA,
            pltpu.SemaphoreType.DMA,
            pltpu.SemaphoreType.DMA,
        ],
        compiler_params=pltpu.CompilerParams(needs_layout_passes=False),
    )(flat, cu_seqlens, zeros_rows)
    return out.reshape(B, MAX_SEQLEN, D)


# speedup vs baseline: 1.1478x; 1.0400x over previous
import jax
import jax.numpy as jnp
from jax import lax
from jax.experimental import pallas as pl
from jax.experimental.pallas import tpu as pltpu
from jax.experimental.pallas import tpu_sc as plsc

B = 8
MAX_SEQLEN = 2048
TOTAL_TOK = 8192
D = 1024

NC = 2
NS = 16
NW = NC * NS
TOTAL_ROWS = B * MAX_SEQLEN
RPW = TOTAL_ROWS // NW
CHUNK = 32
NCHUNK = RPW // CHUNK
NB = 3
ZROWS = 16
LANES = 16


def _body(flat_hbm, cu_hbm, out_hbm, cu_v, zbuf, buf0, buf1, buf2,
          idx_v, idxb_v, sg0, sg1, sg2, ss0, ss1, ss2, sz):
    c = lax.axis_index("c")
    s = lax.axis_index("s")
    b = s // 2
    q = 2 * (s % 2) + ((b + c) % 2)
    j0 = q * RPW
    row0 = b * MAX_SEQLEN + j0

    pltpu.sync_copy(cu_hbm, cu_v.at[pl.ds(0, B + 1)])

    def zfill(r, carry):
        for i in range(D // LANES):
            zbuf[r, pl.ds(i * LANES, LANES)] = jnp.zeros((LANES,), jnp.float32)
        return carry

    lax.fori_loop(0, ZROWS, zfill, 0)

    cu_vec = cu_v[...]
    iota = lax.broadcasted_iota(jnp.int32, (LANES,), 0)
    start = jnp.sum(jnp.where(iota == b, cu_vec, 0))
    end = jnp.sum(jnp.where(iota == b + 1, cu_vec, 0))
    seg_len = jnp.minimum(end - start, MAX_SEQLEN)
    nvalid = jnp.clip(seg_len - j0, 0, RPW)
    kfull = nvalid // CHUNK
    rem = nvalid % CHUNK
    kzero = kfull + (rem > 0).astype(jnp.int32)

    base = start + j0
    bufs = (buf0, buf1, buf2)
    sgs = (sg0, sg1, sg2)
    sss = (ss0, ss1, ss2)

    def out_half(k, h):
        return out_hbm.at[
            pl.ds(pl.multiple_of(row0 + k * CHUNK + h, ZROWS), ZROWS)]

    def out_chunk(k):
        return out_hbm.at[pl.ds(pl.multiple_of(row0 + k * CHUNK, CHUNK), CHUNK)]

    for k in range(NCHUNK):
        @pl.when(k >= kzero)
        def _():
            pltpu.async_copy(zbuf, out_half(k, 0), sz)
            pltpu.async_copy(zbuf, out_half(k, ZROWS), sz)

    for k in range(NCHUNK):
        p = k % NB
        if k >= NB:
            @pl.when(k - NB < kfull)
            def _():
                pltpu.make_async_copy(
                    bufs[p], out_chunk(k - NB), sss[p]).wait()

        for h in range(0, CHUNK, LANES):
            idx_v[k, pl.ds(h, LANES)] = jnp.minimum(
                base + (k * CHUNK + h) + iota, TOTAL_TOK - 1)

        @pl.when(k < kfull)
        def _():
            pltpu.async_copy(flat_hbm.at[idx_v.at[k]], bufs[p], sgs[p])

        if k >= 1:
            q1 = (k - 1) % NB

            @pl.when(k - 1 < kfull)
            def _():
                pltpu.make_async_copy(
                    flat_hbm.at[idx_v.at[k - 1]], bufs[q1], sgs[q1]).wait()
                pltpu.async_copy(bufs[q1], out_chunk(k - 1), sss[q1])

    @pl.when(NCHUNK - 1 < kfull)
    def _():
        q1 = (NCHUNK - 1) % NB
        pltpu.make_async_copy(
            flat_hbm.at[idx_v.at[NCHUNK - 1]], bufs[q1], sgs[q1]).wait()
        pltpu.async_copy(bufs[q1], out_chunk(NCHUNK - 1), sss[q1])

    for k in range(NCHUNK - NB, NCHUNK):
        @pl.when(k < kfull)
        def _():
            pltpu.make_async_copy(
                bufs[k % NB], out_chunk(k), sss[k % NB]).wait()

    bbase = base + kfull * CHUNK
    for h in range(0, CHUNK, LANES):
        idxb_v[pl.ds(h, LANES)] = jnp.minimum(bbase + h + iota, TOTAL_TOK - 1)

    @pl.when(rem > 0)
    def _():
        pltpu.async_copy(flat_hbm.at[idxb_v], buf0, sg0)
        pltpu.make_async_copy(flat_hbm.at[idxb_v], buf0, sg0).wait()

        def zero_row(r, carry):
            for i in range(D // LANES):
                buf0[r, pl.ds(i * LANES, LANES)] = jnp.zeros(
                    (LANES,), jnp.float32)
            return carry

        lax.fori_loop(rem, CHUNK, zero_row, 0)

    bchunk_dst = out_hbm.at[
        pl.ds(pl.multiple_of(row0 + kfull * CHUNK, CHUNK), CHUNK)]

    @pl.when(rem > 0)
    def _():
        pltpu.async_copy(buf0, bchunk_dst, sz)

    for k in range(NCHUNK):
        @pl.when(k >= kzero)
        def _():
            pltpu.make_async_copy(zbuf, out_half(k, 0), sz).wait()
            pltpu.make_async_copy(zbuf, out_half(k, ZROWS), sz).wait()

    @pl.when(rem > 0)
    def _():
        pltpu.make_async_copy(buf0, bchunk_dst, sz).wait()


@jax.jit
def kernel(flat, cu_seqlens):
    mesh = plsc.VectorSubcoreMesh(core_axis_name="c", subcore_axis_name="s")
    out = pl.kernel(
        _body,
        out_type=jax.ShapeDtypeStruct((TOTAL_ROWS, D), jnp.float32),
        mesh=mesh,
        scratch_types=[
            pltpu.VMEM((LANES,), jnp.int32),
            pltpu.VMEM((ZROWS, D), jnp.float32),
            pltpu.VMEM((CHUNK, D), jnp.float32),
            pltpu.VMEM((CHUNK, D), jnp.float32),
            pltpu.VMEM((CHUNK, D), jnp.float32),
            pltpu.VMEM((NCHUNK, CHUNK), jnp.int32),
            pltpu.VMEM((CHUNK,), jnp.int32),
            pltpu.SemaphoreType.DMA,
            pltpu.SemaphoreType.DMA,
            pltpu.SemaphoreType.DMA,
            pltpu.SemaphoreType.DMA,
            pltpu.SemaphoreType.DMA,
            pltpu.SemaphoreType.DMA,
            pltpu.SemaphoreType.DMA,
        ],
        compiler_params=pltpu.CompilerParams(needs_layout_passes=False),
    )(flat, cu_seqlens)
    return out.reshape(B, MAX_SEQLEN, D)
